# baseline (speedup 1.0000x reference)
import jax
import jax.numpy as jnp
from jax import lax
from jax.experimental import pallas as pl
from jax.experimental.pallas import tpu as pltpu
from jax.experimental.pallas import tpu_sc as plsc

NC = 2
NS = 16
NW = NC * NS
LANES = 16

ACTION_SIZE = 1_000_000
FEATURE_DIM = 64
BATCH = 16384
HIST = 200

BBLK = BATCH // NW
BT = BBLK // 128
FG = FEATURE_DIM // 8
TILE_WORDS = 8 * 128
SBUF_WORDS = BBLK * FEATURE_DIM
OUT_WORDS = BATCH * HIST * FEATURE_DIM


def _body(xt_hbm, table_hbm, img_hbm,
          idx0, idx1, g0, g1, sbuf,
          isem0, isem1, gsem0, gsem1, osem):
    wid = lax.axis_index("s") * NC + lax.axis_index("c")
    b0 = wid * BBLK
    iota = lax.iota(jnp.int32, LANES)

    def idx_copy(h, buf, sem):
        pltpu.async_copy(xt_hbm.at[h, pl.ds(b0, BBLK)], buf, sem)

    def wait_idx(buf, sem):
        pltpu.make_async_copy(xt_hbm.at[0, pl.ds(0, BBLK)], buf, sem).wait()

    def gathers(idxbuf, gbuf, sem):
        for j in range(BBLK // 128):
            pltpu.async_copy(
                table_hbm.at[idxbuf.at[pl.ds(j * 128, 128)]],
                gbuf.at[pl.ds(j * 128, 128)],
                sem,
            )

    def wait_gathers(gbuf, sem):
        pltpu.make_async_copy(table_hbm.at[pl.ds(0, BBLK)], gbuf, sem).wait()

    def stage(gbuf):
        @plsc.parallel_loop(0, BT * 8, 1, unroll=2)
        def _tec_body(g):
            bt = g >> 3
            bcg = g & 7
            rowvec = iota + (bt * 128 + bcg * LANES)
            for f in range(FEATURE_DIM):
                colvec = jnp.full((LANES,), f, jnp.int32)
                vals = plsc.load_gather(gbuf, [rowvec, colvec])
                dstoff = (bt * TILE_WORDS + bcg * LANES
                          + ((f // 8) * BT * TILE_WORDS + (f % 8) * 128))
                sbuf[pl.ds(dstoff, LANES)] = vals

    def out_dmas(h):
        for fg in range(FG):
            off = ((h * FG + fg) * (BATCH // 128) + wid * BT) * TILE_WORDS
            pltpu.async_copy(
                sbuf.at[pl.ds(fg * BT * TILE_WORDS, BT * TILE_WORDS)],
                img_hbm.at[pl.ds(off, BT * TILE_WORDS)],
                osem,
            )

    def wait_out():
        pltpu.make_async_copy(img_hbm.at[pl.ds(0, SBUF_WORDS)], sbuf, osem).wait()

    idx_copy(0, idx0, isem0)
    wait_idx(idx0, isem0)
    gathers(idx0, g0, gsem0)
    idx_copy(1, idx1, isem1)

    def step(i, _):
        h0 = 2 * i
        for h, ibuf, isem, gbuf, gsem, oibuf, oisem, ogbuf, ogsem in (
            (h0, idx0, isem0, g0, gsem0, idx1, isem1, g1, gsem1),
            (h0 + 1, idx1, isem1, g1, gsem1, idx0, isem0, g0, gsem0),
        ):
            @pl.when(h < HIST - 1)
            def _():
                wait_idx(oibuf, oisem)
                wait_gathers(gbuf, gsem)
                gathers(oibuf, ogbuf, ogsem)

            @pl.when(h >= HIST - 1)
            def _():
                wait_gathers(gbuf, gsem)

            @pl.when(h < HIST - 2)
            def _():
                idx_copy(h + 2, ibuf, isem)

            @pl.when(h > 0)
            def _():
                wait_out()

            stage(gbuf)
            out_dmas(h)
        return ()

    lax.fori_loop(0, HIST // 2, step, ())
    wait_out()


@jax.jit
def kernel(x, wte):
    xt = x.T.astype(jnp.int32)
    mesh = plsc.VectorSubcoreMesh(core_axis_name="c", subcore_axis_name="s")
    img = pl.kernel(
        _body,
        out_type=jax.ShapeDtypeStruct((OUT_WORDS,), jnp.float32),
        mesh=mesh,
        compiler_params=pltpu.CompilerParams(use_tc_tiling_on_sc=False,
                                             needs_layout_passes=False),
        scratch_types=[
            pltpu.VMEM((BBLK,), jnp.int32),
            pltpu.VMEM((BBLK,), jnp.int32),
            pltpu.VMEM((BBLK, FEATURE_DIM), jnp.float32),
            pltpu.VMEM((BBLK, FEATURE_DIM), jnp.float32),
            pltpu.VMEM((SBUF_WORDS,), jnp.float32),
            pltpu.SemaphoreType.DMA,
            pltpu.SemaphoreType.DMA,
            pltpu.SemaphoreType.DMA,
            pltpu.SemaphoreType.DMA,
            pltpu.SemaphoreType.DMA,
        ],
    )(xt, wte)
    img5 = img.reshape(HIST, FG, BATCH // 128, 8, 128)
    return img5.transpose(2, 4, 0, 1, 3).reshape(BATCH, HIST, FEATURE_DIM)


# speedup vs baseline: 2.1099x; 2.1099x over previous
import jax
import jax.numpy as jnp
from jax import lax
from jax.experimental import pallas as pl
from jax.experimental.pallas import tpu as pltpu
from jax.experimental.pallas import tpu_sc as plsc

NC = 2
NS = 16
NW = NC * NS

ACTION_SIZE = 1_000_000
FEATURE_DIM = 64
BATCH = 16384
HIST = 200

TOT = BATCH * HIST
IDX_MINOR = 128
ROWS = TOT // IDX_MINOR
ROWS_PER_W = ROWS // NW
CHUNK_ROWS = 4
CHUNK = CHUNK_ROWS * IDX_MINOR
OUT_ROWS_PER_CHUNK = CHUNK // 2
N_CHUNKS = ROWS_PER_W // CHUNK_ROWS
N_ITERS = N_CHUNKS // 2


def _body(idx_hbm, table_hbm, out_hbm,
          idx0, idx1, rows0, rows1,
          isem0, isem1, gsem0, gsem1, ssem0, ssem1):
    wid = lax.axis_index("s") * NC + lax.axis_index("c")
    r0 = wid * ROWS_PER_W

    def idx_copy(c, buf, sem):
        pltpu.async_copy(idx_hbm.at[pl.ds(r0 + c * CHUNK_ROWS, CHUNK_ROWS)],
                         buf, sem)

    def wait_idx(buf, sem):
        pltpu.make_async_copy(idx_hbm.at[pl.ds(r0, CHUNK_ROWS)], buf, sem).wait()

    def gathers(idxbuf, rowbuf, sem):
        for j in range(CHUNK_ROWS):
            pltpu.async_copy(
                table_hbm.at[idxbuf.at[j]],
                rowbuf.at[pl.ds(j * IDX_MINOR, IDX_MINOR)],
                sem,
            )

    def wait_gathers(rowbuf, sem):
        pltpu.make_async_copy(table_hbm.at[pl.ds(0, CHUNK)], rowbuf, sem).wait()

    def scatter(c, rowbuf, sem):
        base = (r0 + c * CHUNK_ROWS) * IDX_MINOR
        pltpu.async_copy(
            rowbuf,
            out_hbm.at[pl.ds(base, CHUNK), pl.ds(0, FEATURE_DIM)],
            sem,
        )

    def wait_scatter(rowbuf, sem):
        pltpu.make_async_copy(table_hbm.at[pl.ds(0, CHUNK)], rowbuf, sem).wait()

    idx_copy(0, idx0, isem0)
    wait_idx(idx0, isem0)
    gathers(idx0, rows0, gsem0)
    idx_copy(1, idx1, isem1)

    def step(i, _):
        c0 = 2 * i
        c1 = c0 + 1
        wait_idx(idx1, isem1)

        @pl.when(i > 0)
        def _():
            wait_scatter(rows1, ssem1)

        gathers(idx1, rows1, gsem1)
        wait_gathers(rows0, gsem0)
        scatter(c0, rows0, ssem0)

        @pl.when(i < N_ITERS - 1)
        def _():
            idx_copy(c0 + 2, idx0, isem0)

        wait_scatter(rows0, ssem0)

        @pl.when(i < N_ITERS - 1)
        def _():
            wait_idx(idx0, isem0)
            gathers(idx0, rows0, gsem0)

        wait_gathers(rows1, gsem1)
        scatter(c1, rows1, ssem1)

        @pl.when(i < N_ITERS - 1)
        def _():
            idx_copy(c1 + 2, idx1, isem1)

        return ()

    lax.fori_loop(0, N_ITERS, step, ())

    wait_scatter(rows1, ssem1)


@jax.jit
def kernel(x, wte):
    xf = x.reshape(ROWS, IDX_MINOR).astype(jnp.int32)
    mesh = plsc.VectorSubcoreMesh(core_axis_name="c", subcore_axis_name="s")
    out = pl.kernel(
        _body,
        out_type=jax.ShapeDtypeStruct((TOT, 2 * FEATURE_DIM), jnp.float32),
        mesh=mesh,
        compiler_params=pltpu.CompilerParams(use_tc_tiling_on_sc=False),
        scratch_types=[
            pltpu.VMEM((CHUNK_ROWS, IDX_MINOR), jnp.int32),
            pltpu.VMEM((CHUNK_ROWS, IDX_MINOR), jnp.int32),
            pltpu.VMEM((CHUNK, FEATURE_DIM), jnp.float32),
            pltpu.VMEM((CHUNK, FEATURE_DIM), jnp.float32),
            pltpu.SemaphoreType.DMA,
            pltpu.SemaphoreType.DMA,
            pltpu.SemaphoreType.DMA,
            pltpu.SemaphoreType.DMA,
            pltpu.SemaphoreType.DMA,
            pltpu.SemaphoreType.DMA,
        ],
    )(xf, wte)
    return out[:, :FEATURE_DIM].reshape(BATCH, HIST, FEATURE_DIM)


# speedup vs baseline: 2.1117x; 1.0008x over previous
import jax
import jax.numpy as jnp
from jax import lax
from jax.experimental import pallas as pl
from jax.experimental.pallas import tpu as pltpu
from jax.experimental.pallas import tpu_sc as plsc

NC = 2
NS = 16
NW = NC * NS

ACTION_SIZE = 1_000_000
FEATURE_DIM = 64
BATCH = 16384
HIST = 200

TOT = BATCH * HIST
IDX_MINOR = 128
ROWS = TOT // IDX_MINOR
ROWS_PER_W = ROWS // NW
CHUNK_ROWS = 5
CHUNK = CHUNK_ROWS * IDX_MINOR
OUT_ROWS_PER_CHUNK = CHUNK // 2
N_CHUNKS = ROWS_PER_W // CHUNK_ROWS
N_ITERS = N_CHUNKS // 2


def _body(idx_hbm, table_hbm, out_hbm,
          idx0, idx1, rows0, rows1,
          isem0, isem1, gsem0, gsem1, ssem0, ssem1):
    wid = lax.axis_index("s") * NC + lax.axis_index("c")
    r0 = wid * ROWS_PER_W

    def idx_copy(c, buf, sem):
        pltpu.async_copy(idx_hbm.at[pl.ds(r0 + c * CHUNK_ROWS, CHUNK_ROWS)],
                         buf, sem)

    def wait_idx(buf, sem):
        pltpu.make_async_copy(idx_hbm.at[pl.ds(r0, CHUNK_ROWS)], buf, sem).wait()

    def gathers(idxbuf, rowbuf, sem):
        for j in range(CHUNK_ROWS):
            pltpu.async_copy(
                table_hbm.at[idxbuf.at[j]],
                rowbuf.at[pl.ds(j * IDX_MINOR, IDX_MINOR)],
                sem,
            )

    def wait_gathers(rowbuf, sem):
        pltpu.make_async_copy(table_hbm.at[pl.ds(0, CHUNK)], rowbuf, sem).wait()

    def scatter(c, rowbuf, sem):
        base = (r0 + c * CHUNK_ROWS) * IDX_MINOR
        pltpu.async_copy(
            rowbuf,
            out_hbm.at[pl.ds(base, CHUNK), pl.ds(0, FEATURE_DIM)],
            sem,
        )

    def wait_scatter(rowbuf, sem):
        pltpu.make_async_copy(table_hbm.at[pl.ds(0, CHUNK)], rowbuf, sem).wait()

    idx_copy(0, idx0, isem0)
    wait_idx(idx0, isem0)
    gathers(idx0, rows0, gsem0)
    idx_copy(1, idx1, isem1)

    def step(i, _):
        c0 = 2 * i
        c1 = c0 + 1
        wait_idx(idx1, isem1)

        @pl.when(i > 0)
        def _():
            wait_scatter(rows1, ssem1)

        gathers(idx1, rows1, gsem1)
        wait_gathers(rows0, gsem0)
        scatter(c0, rows0, ssem0)

        @pl.when(i < N_ITERS - 1)
        def _():
            idx_copy(c0 + 2, idx0, isem0)

        wait_scatter(rows0, ssem0)

        @pl.when(i < N_ITERS - 1)
        def _():
            wait_idx(idx0, isem0)
            gathers(idx0, rows0, gsem0)

        wait_gathers(rows1, gsem1)
        scatter(c1, rows1, ssem1)

        @pl.when(i < N_ITERS - 1)
        def _():
            idx_copy(c1 + 2, idx1, isem1)

        return ()

    lax.fori_loop(0, N_ITERS, step, ())

    wait_scatter(rows1, ssem1)


@jax.jit
def kernel(x, wte):
    xf = x.reshape(ROWS, IDX_MINOR).astype(jnp.int32)
    mesh = plsc.VectorSubcoreMesh(core_axis_name="c", subcore_axis_name="s")
    out = pl.kernel(
        _body,
        out_type=jax.ShapeDtypeStruct((TOT, 2 * FEATURE_DIM), jnp.float32),
        mesh=mesh,
        compiler_params=pltpu.CompilerParams(use_tc_tiling_on_sc=False),
        scratch_types=[
            pltpu.VMEM((CHUNK_ROWS, IDX_MINOR), jnp.int32),
            pltpu.VMEM((CHUNK_ROWS, IDX_MINOR), jnp.int32),
            pltpu.VMEM((CHUNK, FEATURE_DIM), jnp.float32),
            pltpu.VMEM((CHUNK, FEATURE_DIM), jnp.float32),
            pltpu.SemaphoreType.DMA,
            pltpu.SemaphoreType.DMA,
            pltpu.SemaphoreType.DMA,
            pltpu.SemaphoreType.DMA,
            pltpu.SemaphoreType.DMA,
            pltpu.SemaphoreType.DMA,
        ],
    )(xf, wte)
    return out[:, :FEATURE_DIM].reshape(BATCH, HIST, FEATURE_DIM)
